# baseline (speedup 1.0000x reference)
import functools
import jax
import jax.numpy as jnp
from jax import lax
from jax.experimental import pallas as pl
from jax.experimental.pallas import tpu as pltpu
from jax.experimental.pallas import tpu_sc as plsc

H = 32
W = 32
D = 256
NCORES = 2
NSUB = 16
HPS = H // NSUB
BPC = 32 // NCORES
L = 16


def _sc_body(col_hbm, row_hbm, out_hbm, col_v, row_v, buf_v, sem, dsem):
    cid = lax.axis_index("c")
    sid = lax.axis_index("s")
    h0 = sid * HPS

    cp1 = pltpu.async_copy(col_hbm.at[pl.ds(0, W), :], col_v, sem)
    cp2 = pltpu.async_copy(row_hbm.at[pl.ds(h0, HPS), :], row_v, sem)
    cp1.wait()
    cp2.wait()

    def build_w(w, _):
        for hl in range(HPS):
            for k in range(D // L):
                buf_v[hl, w, pl.ds(k * L, L)] = col_v[w, pl.ds(k * L, L)]
                buf_v[hl, w, pl.ds(D + k * L, L)] = row_v[hl, pl.ds(k * L, L)]
        return 0

    lax.fori_loop(0, W, build_w, 0)

    copies = []
    for j in range(BPC):
        b = cid * BPC + j
        copies.append(
            pltpu.async_copy(buf_v, out_hbm.at[b, pl.ds(h0, HPS), :, :], dsem))
    for cp in copies:
        cp.wait()


def kernel(x, row_embed, col_embed):
    batch = x.shape[0]
    mesh = plsc.VectorSubcoreMesh(core_axis_name="c", subcore_axis_name="s")
    run = functools.partial(
        pl.kernel,
        out_type=jax.ShapeDtypeStruct((batch, H, W, 2 * D), jnp.float32),
        mesh=mesh,
        scratch_types=[
            pltpu.VMEM((W, D), jnp.float32),
            pltpu.VMEM((HPS, D), jnp.float32),
            pltpu.VMEM((HPS, W, 2 * D), jnp.float32),
            pltpu.SemaphoreType.DMA,
            pltpu.SemaphoreType.DMA,
        ],
        compiler_params=pltpu.CompilerParams(
            needs_layout_passes=False, use_tc_tiling_on_sc=True),
    )(_sc_body)
    out_cm = run(col_embed, row_embed)
    return jnp.transpose(out_cm, (0, 3, 1, 2))


# speedup vs baseline: 1.0112x; 1.0112x over previous
import functools
import jax
import jax.numpy as jnp
from jax import lax
from jax.experimental import pallas as pl
from jax.experimental.pallas import tpu as pltpu
from jax.experimental.pallas import tpu_sc as plsc

H = 32
W = 32
D = 256
NCORES = 2
NSUB = 16
HPS = H // NSUB
BPC = 32 // NCORES
L = 16


def _sc_body(col_hbm, row_hbm, out_hbm, col_v, row_v, buf_v, sem, dsem):
    cid = lax.axis_index("c")
    sid = lax.axis_index("s")
    h0 = sid * HPS

    cp1 = pltpu.async_copy(col_hbm.at[pl.ds(0, W), :], col_v, sem)
    cp2 = pltpu.async_copy(row_hbm.at[pl.ds(h0, HPS), :], row_v, sem)
    cp1.wait()
    cp2.wait()

    for hl in range(HPS):
        row_regs = [row_v[hl, pl.ds(k * L, L)] for k in range(D // L)]
        for w in range(W):
            for k in range(D // L):
                buf_v[hl, w, pl.ds(k * L, L)] = col_v[w, pl.ds(k * L, L)]
                buf_v[hl, w, pl.ds(D + k * L, L)] = row_regs[k]

    copies = []
    for j in range(BPC):
        b = cid * BPC + j
        copies.append(
            pltpu.async_copy(buf_v, out_hbm.at[b, pl.ds(h0, HPS), :, :], dsem))
    for cp in copies:
        cp.wait()


def kernel(x, row_embed, col_embed):
    batch = x.shape[0]
    mesh = plsc.VectorSubcoreMesh(core_axis_name="c", subcore_axis_name="s")
    run = functools.partial(
        pl.kernel,
        out_type=jax.ShapeDtypeStruct((batch, H, W, 2 * D), jnp.float32),
        mesh=mesh,
        scratch_types=[
            pltpu.VMEM((W, D), jnp.float32),
            pltpu.VMEM((HPS, D), jnp.float32),
            pltpu.VMEM((HPS, W, 2 * D), jnp.float32),
            pltpu.SemaphoreType.DMA,
            pltpu.SemaphoreType.DMA,
        ],
        compiler_params=pltpu.CompilerParams(
            needs_layout_passes=False, use_tc_tiling_on_sc=True,
            skip_device_barrier=True),
    )(_sc_body)
    out_cm = run(col_embed, row_embed)
    return jnp.transpose(out_cm, (0, 3, 1, 2))


# speedup vs baseline: 1.0145x; 1.0033x over previous
import functools
import jax
import jax.numpy as jnp
from jax import lax
from jax.experimental import pallas as pl
from jax.experimental.pallas import tpu as pltpu
from jax.experimental.pallas import tpu_sc as plsc

H = 32
W = 32
D = 256
NCORES = 2
NSUB = 16
HPS = H // NSUB
BPC = 32 // NCORES
L = 16


def _sc_body(col_hbm, row_hbm, out_hbm, col_v, row_v, buf_v, sem, dsem):
    cid = lax.axis_index("c")
    sid = lax.axis_index("s")
    h0 = sid * HPS

    cp1 = pltpu.async_copy(col_hbm.at[pl.ds(0, W), :], col_v, sem)
    cp2 = pltpu.async_copy(row_hbm.at[pl.ds(h0, HPS), :], row_v, sem)
    cp1.wait()
    cp2.wait()

    for hl in range(HPS):
        row_regs = [row_v[hl, pl.ds(k * L, L)] for k in range(D // L)]
        for w in range(W):
            for k in range(D // L):
                buf_v[hl, w, pl.ds(k * L, L)] = col_v[w, pl.ds(k * L, L)]
                buf_v[hl, w, pl.ds(D + k * L, L)] = row_regs[k]

    copies = []
    for j in range(BPC):
        b = cid * BPC + j
        copies.append(
            pltpu.async_copy(buf_v, out_hbm.at[b, pl.ds(h0, HPS), :, :], dsem))
    for cp in copies:
        cp.wait()


def kernel(x, row_embed, col_embed):
    batch = x.shape[0]
    mesh = plsc.VectorSubcoreMesh(core_axis_name="c", subcore_axis_name="s")
    run = functools.partial(
        pl.kernel,
        out_type=jax.ShapeDtypeStruct((batch, H, W, 2 * D), jnp.float32),
        mesh=mesh,
        scratch_types=[
            pltpu.VMEM((W, D), jnp.float32),
            pltpu.VMEM((HPS, D), jnp.float32),
            pltpu.VMEM((HPS, W, 2 * D), jnp.float32),
            pltpu.SemaphoreType.DMA,
            pltpu.SemaphoreType.DMA,
        ],
        compiler_params=pltpu.CompilerParams(
            needs_layout_passes=False, use_tc_tiling_on_sc=True),
    )(_sc_body)
    out_cm = run(col_embed, row_embed)
    return jnp.transpose(out_cm, (0, 3, 1, 2))
